# baseline (speedup 1.0000x reference)
import jax
import jax.numpy as jnp
from jax import lax
from jax.experimental import pallas as pl
from jax.experimental.pallas import tpu as pltpu
from jax.experimental.pallas import tpu_sc as plsc

_N = 10000
_E = 320000
_D = 128

_NC = 2
_NS = 16
_HD = _D // _NC
_EPT = _E // _NS
_K = 80
_KG = _K // 8
_CH = _EPT // _K
_RPT = 624
_TAIL = _N - _NS * _RPT


def _sc_body(m_hbm, edge_hbm, vals_hbm, zeros_hbm, out_hbm,
             colv, rowv, valv, grows, sbuf, acc, sg0, sg1, ss0, ss1):
    cid = lax.axis_index("c")
    sid = lax.axis_index("s")
    sg = (sg0, sg1)
    ss = (ss0, ss1)

    pltpu.sync_copy(edge_hbm.at[1, sid], colv)
    pltpu.sync_copy(edge_hbm.at[0, sid], rowv)
    pltpu.sync_copy(vals_hbm.at[pl.ds(sid * _EPT, _EPT)], valv)
    pltpu.sync_copy(zeros_hbm.at[pl.ds(sid * _RPT, _RPT)],
                    acc.at[pl.ds(sid * _RPT, _RPT)])

    @pl.when(sid == 0)
    def _():
        pltpu.sync_copy(zeros_hbm.at[pl.ds(_NS * _RPT, _TAIL)],
                        acc.at[pl.ds(_NS * _RPT, _TAIL)])

    base = jnp.full((16,), cid, jnp.int32)

    def adj_body(i, carry):
        for r in range(_K // 16):
            sl = pl.ds(r * 16, 16)
            cv = colv[i, sl]
            colv[i, sl] = cv + cv + base
        return carry

    lax.fori_loop(0, _CH, adj_body, 0)
    plsc.subcore_barrier()

    def _issue(c, b):
        pltpu.async_copy(m_hbm.at[colv.at[c]], grows.at[b], sg[b])

    def _wait(c, b):
        pltpu.make_async_copy(m_hbm.at[colv.at[c]], grows.at[b], sg[b]).wait()

    _issue(0, 0)
    _issue(1, 1)

    def chunk_pair(c2, carry):
        for b in range(2):
            c = 2 * c2 + b
            _wait(c, b)

            @pl.when(c2 >= 1)
            def _():
                pltpu.make_async_copy(
                    sbuf.at[b], acc.at[rowv.at[c]], ss[b]).wait()

            def edge_body(g, carry2):
                nr = _HD // 16
                vals16 = valv[pl.ds(c * _K + g * 16, 16)]
                dnums = lax.GatherDimensionNumbers(
                    offset_dims=(), collapsed_slice_dims=(0,),
                    start_index_map=(0,))
                for h in range(2):
                    vvs = [lax.gather(
                        vals16, jnp.full((16, 1), h * 8 + u, jnp.int32),
                        dnums, slice_sizes=(1,),
                        mode=lax.GatherScatterMode.PROMISE_IN_BOUNDS)
                           for u in range(8)]
                    gvs = [[grows[b, g * 16 + h * 8 + u, pl.ds(r * 16, 16)]
                            for r in range(nr)] for u in range(8)]
                    for u in range(8):
                        for r in range(nr):
                            sbuf[b, g * 16 + h * 8 + u, pl.ds(r * 16, 16)] = (
                                gvs[u][r] * vvs[u])
                return carry2

            lax.fori_loop(0, _K // 16, edge_body, 0)
            pltpu.async_copy(sbuf.at[b], acc.at[rowv.at[c]], ss[b], add=True)

            @pl.when(c + 2 < _CH)
            def _():
                _issue(c + 2, b)

        return carry

    lax.fori_loop(0, _CH // 2, chunk_pair, 0)
    for b in range(2):
        pltpu.make_async_copy(sbuf.at[b], acc.at[rowv.at[b]], ss[b]).wait()
    plsc.subcore_barrier()
    pltpu.sync_copy(acc.at[pl.ds(sid * _RPT, _RPT)],
                    out_hbm.at[pl.ds(sid * _RPT, _RPT), cid])

    @pl.when(sid == 0)
    def _():
        pltpu.sync_copy(acc.at[pl.ds(_NS * _RPT, _TAIL)],
                        out_hbm.at[pl.ds(_NS * _RPT, _TAIL), cid])


_sc_spmm = pl.kernel(
    _sc_body,
    out_type=jax.ShapeDtypeStruct((_N, _NC, _HD), jnp.float32),
    mesh=plsc.VectorSubcoreMesh(core_axis_name="c", subcore_axis_name="s"),
    compiler_params=pltpu.CompilerParams(use_tc_tiling_on_sc=False),
    scratch_types=[
        pltpu.VMEM((_CH, _K), jnp.int32),
        pltpu.VMEM((_CH, _K), jnp.int32),
        pltpu.VMEM((_EPT,), jnp.float32),
        pltpu.VMEM((2, _K, _HD), jnp.float32),
        pltpu.VMEM((2, _K, _HD), jnp.float32),
        pltpu.VMEM_SHARED((_N, _HD), jnp.float32),
        pltpu.SemaphoreType.DMA,
        pltpu.SemaphoreType.DMA,
        pltpu.SemaphoreType.DMA,
        pltpu.SemaphoreType.DMA,
    ],
)


def kernel(m, edge_index, edge_vals):
    edges = edge_index.reshape(2, _NS, _CH, _K)
    m_split = m.astype(jnp.float32).reshape(2 * _N, _HD)
    zeros = jnp.zeros((_N, _HD), jnp.float32)
    out = _sc_spmm(m_split, edges, edge_vals.astype(jnp.float32),
                   zeros)
    return out.reshape(_N, _D).astype(m.dtype)


# speedup vs baseline: 1.2258x; 1.2258x over previous
import jax
import jax.numpy as jnp
from jax import lax
from jax.experimental import pallas as pl
from jax.experimental.pallas import tpu as pltpu
from jax.experimental.pallas import tpu_sc as plsc

_N = 10000
_E = 320000
_D = 128

_NC = 2
_NS = 16
_HD = _D // _NC
_EPT = _E // _NS
_K = 80
_KG = _K // 8
_CH = _EPT // _K
_RPT = 624
_TAIL = _N - _NS * _RPT


def _sc_body(m_hbm, edge_hbm, vals_hbm, zeros_hbm, out_hbm,
             colv, rowv, valv, grows, sbuf, acc, sg0, sg1, ss0, ss1):
    cid = lax.axis_index("c")
    sid = lax.axis_index("s")
    sg = (sg0, sg1)
    ss = (ss0, ss1)

    pltpu.sync_copy(edge_hbm.at[1, sid], colv)
    pltpu.sync_copy(edge_hbm.at[0, sid], rowv)
    pltpu.sync_copy(vals_hbm.at[pl.ds(sid * _EPT, _EPT)], valv)
    pltpu.sync_copy(zeros_hbm.at[pl.ds(sid * _RPT, _RPT)],
                    acc.at[pl.ds(sid * _RPT, _RPT)])

    @pl.when(sid == 0)
    def _():
        pltpu.sync_copy(zeros_hbm.at[pl.ds(_NS * _RPT, _TAIL)],
                        acc.at[pl.ds(_NS * _RPT, _TAIL)])

    base = jnp.full((16,), cid, jnp.int32)

    def adj_body(i, carry):
        for r in range(_K // 16):
            sl = pl.ds(r * 16, 16)
            cv = colv[i, sl]
            colv[i, sl] = cv + cv + base
        return carry

    lax.fori_loop(0, _CH, adj_body, 0)
    plsc.subcore_barrier()

    def _issue(c, b):
        pltpu.async_copy(m_hbm.at[colv.at[c]], grows.at[b], sg[b])

    def _wait(c, b):
        pltpu.make_async_copy(m_hbm.at[colv.at[c]], grows.at[b], sg[b]).wait()

    _issue(0, 0)
    _issue(1, 1)

    def chunk_pair(c2, carry):
        for b in range(2):
            c = 2 * c2 + b
            _wait(c, b)

            @pl.when(c2 >= 1)
            def _():
                pltpu.make_async_copy(
                    sbuf.at[b], acc.at[rowv.at[c]], ss[b]).wait()

            def edge_body(g, carry2):
                nr = _HD // 16
                vals16 = valv[pl.ds(c * _K + g * 16, 16)]
                dnums = lax.GatherDimensionNumbers(
                    offset_dims=(), collapsed_slice_dims=(0,),
                    start_index_map=(0,))
                for h in range(2):
                    vvs = [lax.gather(
                        vals16, jnp.full((16, 1), h * 8 + u, jnp.int32),
                        dnums, slice_sizes=(1,),
                        mode=lax.GatherScatterMode.PROMISE_IN_BOUNDS)
                           for u in range(8)]
                    gvs = [[grows[b, g * 16 + h * 8 + u, pl.ds(r * 16, 16)]
                            for r in range(nr)] for u in range(8)]
                    for u in range(8):
                        for r in range(nr):
                            sbuf[b, g * 16 + h * 8 + u, pl.ds(r * 16, 16)] = (
                                gvs[u][r] * vvs[u])
                return carry2

            lax.fori_loop(0, _K // 16, edge_body, 0)
            pltpu.async_copy(sbuf.at[b], acc.at[rowv.at[c]], ss[b], add=True)

            @pl.when(c + 2 < _CH)
            def _():
                _issue(c + 2, b)

        return carry

    lax.fori_loop(0, _CH // 2, chunk_pair, 0)
    for b in range(2):
        pltpu.make_async_copy(sbuf.at[b], acc.at[rowv.at[b]], ss[b]).wait()
    plsc.subcore_barrier()
    pltpu.sync_copy(acc.at[pl.ds(sid * _RPT, _RPT)],
                    out_hbm.at[pl.ds(sid * _RPT, _RPT), pl.ds(cid * _HD, _HD)])

    @pl.when(sid == 0)
    def _():
        pltpu.sync_copy(
            acc.at[pl.ds(_NS * _RPT, _TAIL)],
            out_hbm.at[pl.ds(_NS * _RPT, _TAIL), pl.ds(cid * _HD, _HD)])


_sc_spmm = pl.kernel(
    _sc_body,
    out_type=jax.ShapeDtypeStruct((_N, _D), jnp.float32),
    mesh=plsc.VectorSubcoreMesh(core_axis_name="c", subcore_axis_name="s"),
    compiler_params=pltpu.CompilerParams(use_tc_tiling_on_sc=False),
    scratch_types=[
        pltpu.VMEM((_CH, _K), jnp.int32),
        pltpu.VMEM((_CH, _K), jnp.int32),
        pltpu.VMEM((_EPT,), jnp.float32),
        pltpu.VMEM((2, _K, _HD), jnp.float32),
        pltpu.VMEM((2, _K, _HD), jnp.float32),
        pltpu.VMEM_SHARED((_N, _HD), jnp.float32),
        pltpu.SemaphoreType.DMA,
        pltpu.SemaphoreType.DMA,
        pltpu.SemaphoreType.DMA,
        pltpu.SemaphoreType.DMA,
    ],
)


def kernel(m, edge_index, edge_vals):
    edges = edge_index.reshape(2, _NS, _CH, _K)
    m_split = m.astype(jnp.float32).reshape(2 * _N, _HD)
    zeros = jnp.zeros((_N, _HD), jnp.float32)
    out = _sc_spmm(m_split, edges, edge_vals.astype(jnp.float32), zeros)
    return out.astype(m.dtype)


# speedup vs baseline: 1.2275x; 1.0014x over previous
import jax
import jax.numpy as jnp
from jax import lax
from jax.experimental import pallas as pl
from jax.experimental.pallas import tpu as pltpu
from jax.experimental.pallas import tpu_sc as plsc

_N = 10000
_E = 320000
_D = 128

_NC = 2
_NS = 16
_HD = _D // _NC
_EPT = _E // _NS
_K = 80
_CH = _EPT // _K
_RPT = 624
_TAIL = _N - _NS * _RPT


def _sc_body(m_hbm, edge_hbm, vals_hbm, zeros_hbm, out_hbm,
             colv, rowv, valv, grows, sbuf, acc, sg0, sg1, ss0, ss1):
    cid = lax.axis_index("c")
    sid = lax.axis_index("s")
    sg = (sg0, sg1)
    ss = (ss0, ss1)

    pltpu.sync_copy(edge_hbm.at[1, sid], colv)
    pltpu.sync_copy(edge_hbm.at[0, sid], rowv)
    pltpu.sync_copy(vals_hbm.at[pl.ds(sid * _EPT, _EPT)], valv)
    pltpu.sync_copy(zeros_hbm.at[pl.ds(sid * _RPT, _RPT)],
                    acc.at[pl.ds(sid * _RPT, _RPT)])

    @pl.when(sid == 0)
    def _():
        pltpu.sync_copy(zeros_hbm.at[pl.ds(_NS * _RPT, _TAIL)],
                        acc.at[pl.ds(_NS * _RPT, _TAIL)])

    base = jnp.full((16,), cid, jnp.int32)

    def adj_body(i, carry):
        for r in range(_K // 16):
            sl = pl.ds(r * 16, 16)
            cv = colv[i, sl]
            colv[i, sl] = cv + cv + base
        return carry

    lax.fori_loop(0, _CH, adj_body, 0)
    plsc.subcore_barrier()

    def _issue(c, b):
        pltpu.async_copy(m_hbm.at[colv.at[c]], grows.at[b], sg[b])

    def _wait(c, b):
        pltpu.make_async_copy(m_hbm.at[colv.at[c]], grows.at[b], sg[b]).wait()

    _issue(0, 0)
    _issue(1, 1)

    def chunk_pair(c2, carry):
        for b in range(2):
            c = 2 * c2 + b
            _wait(c, b)

            @pl.when(c2 >= 1)
            def _():
                pltpu.make_async_copy(
                    sbuf.at[b], acc.at[rowv.at[c]], ss[b]).wait()

            def edge_body(g, carry2):
                nr = _HD // 16
                vals16 = valv[pl.ds(c * _K + g * 16, 16)]
                dnums = lax.GatherDimensionNumbers(
                    offset_dims=(), collapsed_slice_dims=(0,),
                    start_index_map=(0,))
                for h in range(2):
                    vvs = [lax.gather(
                        vals16, jnp.full((16, 1), h * 8 + u, jnp.int32),
                        dnums, slice_sizes=(1,),
                        mode=lax.GatherScatterMode.PROMISE_IN_BOUNDS)
                           for u in range(8)]
                    gvs = [[grows[b, g * 16 + h * 8 + u, pl.ds(r * 16, 16)]
                            for r in range(nr)] for u in range(8)]
                    for u in range(8):
                        for r in range(nr):
                            sbuf[b, g * 16 + h * 8 + u, pl.ds(r * 16, 16)] = (
                                gvs[u][r] * vvs[u])
                return carry2

            lax.fori_loop(0, _K // 16, edge_body, 0)
            pltpu.async_copy(sbuf.at[b], acc.at[rowv.at[c]], ss[b], add=True)

            @pl.when(c + 2 < _CH)
            def _():
                _issue(c + 2, b)

        return carry

    lax.fori_loop(0, _CH // 2, chunk_pair, 0)
    for b in range(2):
        pltpu.make_async_copy(sbuf.at[b], acc.at[rowv.at[b]], ss[b]).wait()
    plsc.subcore_barrier()
    pltpu.sync_copy(acc.at[pl.ds(sid * _RPT, _RPT)],
                    out_hbm.at[pl.ds(sid * _RPT, _RPT), pl.ds(cid * _HD, _HD)])

    @pl.when(sid == 0)
    def _():
        pltpu.sync_copy(
            acc.at[pl.ds(_NS * _RPT, _TAIL)],
            out_hbm.at[pl.ds(_NS * _RPT, _TAIL), pl.ds(cid * _HD, _HD)])


_sc_spmm = pl.kernel(
    _sc_body,
    out_type=jax.ShapeDtypeStruct((_N, _D), jnp.float32),
    mesh=plsc.VectorSubcoreMesh(core_axis_name="c", subcore_axis_name="s"),
    compiler_params=pltpu.CompilerParams(use_tc_tiling_on_sc=False),
    scratch_types=[
        pltpu.VMEM((_CH, _K), jnp.int32),
        pltpu.VMEM((_CH, _K), jnp.int32),
        pltpu.VMEM((_EPT,), jnp.float32),
        pltpu.VMEM((2, _K, _HD), jnp.float32),
        pltpu.VMEM((2, _K, _HD), jnp.float32),
        pltpu.VMEM_SHARED((_N, _HD), jnp.float32),
        pltpu.SemaphoreType.DMA,
        pltpu.SemaphoreType.DMA,
        pltpu.SemaphoreType.DMA,
        pltpu.SemaphoreType.DMA,
    ],
)


def kernel(m, edge_index, edge_vals):
    edges = edge_index.reshape(2, _NS, _CH, _K)
    m_split = m.astype(jnp.float32).reshape(2 * _N, _HD)
    zeros = jnp.zeros((_N, _HD), jnp.float32)
    out = _sc_spmm(m_split, edges, edge_vals.astype(jnp.float32), zeros)
    return out.astype(m.dtype)
